# Initial kernel scaffold; baseline (speedup 1.0000x reference)
import jax
import jax.numpy as jnp
from jax.experimental import pallas as pl


def kernel(x, edge_index, W, att_src, att_dst, bias):
    raise NotImplementedError("write your pallas kernel here")


# speedup vs baseline: 1.0543x; 1.0543x over previous
import jax
import jax.numpy as jnp
from jax.experimental import pallas as pl

IN_CHANNELS = 128
OUT_CHANNELS = 32
HEADS = 4
N_NODES = 10000
N_EDGES = 320000


def _proj_body(x_ref, w_ref, s_ref, xp_ref, a8_ref):
    xp = jnp.dot(x_ref[...], w_ref[...], preferred_element_type=jnp.float32)
    xp_ref[...] = xp
    a8_ref[...] = jnp.dot(xp, s_ref[...], preferred_element_type=jnp.float32)


def kernel(x, edge_index, W, att_src, att_dst, bias):
    N = x.shape[0]
    H, C = HEADS, OUT_CHANNELS
    eye = jnp.eye(H, dtype=jnp.float32)
    s_src = (att_src[:, None, :] * eye[:, :, None]).reshape(H * C, H)
    s_src = (eye[:, :, None] * att_src[:, None, :]).transpose(0, 2, 1).reshape(H * C, H)
    s_dst = (eye[:, :, None] * att_dst[:, None, :]).transpose(0, 2, 1).reshape(H * C, H)
    S = jnp.concatenate([s_src, s_dst], axis=1)
    S = jnp.pad(S, ((0, 0), (0, 128 - 2 * H)))

    blk = 1000
    grid = N // blk
    xp, a8 = pl.pallas_call(
        _proj_body,
        grid=(grid,),
        in_specs=[
            pl.BlockSpec((blk, IN_CHANNELS), lambda i: (i, 0)),
            pl.BlockSpec((IN_CHANNELS, H * C), lambda i: (0, 0)),
            pl.BlockSpec((H * C, 128), lambda i: (0, 0)),
        ],
        out_specs=(
            pl.BlockSpec((blk, H * C), lambda i: (i, 0)),
            pl.BlockSpec((blk, 128), lambda i: (i, 0)),
        ),
        out_shape=(
            jax.ShapeDtypeStruct((N, H * C), jnp.float32),
            jax.ShapeDtypeStruct((N, 128), jnp.float32),
        ),
    )(x, W, S)

    a_src = a8[:, :H]
    a_dst = a8[:, H:2 * H]
    src = edge_index[0]
    dst = edge_index[1]
    alpha = a_src[src] + a_dst[dst]
    alpha = jax.nn.leaky_relu(alpha, negative_slope=0.2)
    ex = jnp.exp(alpha)
    denom = jax.ops.segment_sum(ex, dst, num_segments=N)
    coef = ex / (denom[dst] + 1e-16)
    xph = xp.reshape(N, H, C)
    msg = xph[src] * coef[:, :, None]
    out = jax.ops.segment_sum(msg, dst, num_segments=N)
    return out.reshape(N, H * C) + bias[None, :]
